# baseline (speedup 1.0000x reference)
import jax
import jax.numpy as jnp
from jax import lax
from jax.experimental import pallas as pl
from jax.experimental.pallas import tpu as pltpu
from jax.experimental.pallas import tpu_sc as plsc

B = 16384
D = 16
NC = 2
NS = 16
L = 16
NW = NC * NS
BPW = B // NW
NBLK = BPW // L


def _cmf_body(uid_hbm, iid_hbm, utab_hbm, itab_hbm, out_hbm,
              uid_v, iid_v, urows_v, irows_v, out_v, sem_u, sem_i):
    wid = lax.axis_index("s") * NC + lax.axis_index("c")
    base = wid * BPW

    pltpu.sync_copy(uid_hbm.at[pl.ds(base, BPW)], uid_v)
    pltpu.sync_copy(iid_hbm.at[pl.ds(base, BPW)], iid_v)
    cu = pltpu.async_copy(utab_hbm.at[uid_v], urows_v, sem_u)
    ci = pltpu.async_copy(itab_hbm.at[iid_v], irows_v, sem_i)
    cu.wait()
    ci.wait()

    row0 = lax.iota(jnp.int32, L)
    for b in range(NBLK):
        rows = row0 + b * L
        acc = jnp.zeros((L,), jnp.float32)
        for d in range(D):
            col = jnp.full((L,), d, jnp.int32)
            u = plsc.load_gather(urows_v, [rows, col])
            it = plsc.load_gather(irows_v, [rows, col])
            acc = acc + u * it
        out_v[pl.ds(b * L, L)] = 1.0 / (1.0 + jnp.exp(-acc))

    pltpu.sync_copy(out_v, out_hbm.at[pl.ds(base, BPW)])


def kernel(user_ids, item_ids, source_user, source_item):
    mesh = plsc.VectorSubcoreMesh(
        core_axis_name="c", subcore_axis_name="s",
        num_cores=NC, num_subcores=NS)
    k = pl.kernel(
        _cmf_body,
        out_type=jax.ShapeDtypeStruct((B,), jnp.float32),
        mesh=mesh,
        compiler_params=pltpu.CompilerParams(
            needs_layout_passes=False, use_tc_tiling_on_sc=False),
        scratch_types=[
            pltpu.VMEM((BPW,), jnp.int32),
            pltpu.VMEM((BPW,), jnp.int32),
            pltpu.VMEM((BPW, D), jnp.float32),
            pltpu.VMEM((BPW, D), jnp.float32),
            pltpu.VMEM((BPW,), jnp.float32),
            pltpu.SemaphoreType.DMA,
            pltpu.SemaphoreType.DMA,
        ],
    )
    return k(user_ids.astype(jnp.int32), item_ids.astype(jnp.int32),
             source_user, source_item)


# speedup vs baseline: 5.7993x; 5.7993x over previous
import jax
import jax.numpy as jnp
from jax import lax
from jax.experimental import pallas as pl
from jax.experimental.pallas import tpu as pltpu
from jax.experimental.pallas import tpu_sc as plsc

B = 16384
D = 16
NC = 2
NS = 16
L = 16
NW = NC * NS
BPW = B // NW
CHUNK = 16
NCHUNK = BPW // CHUNK


def _cmf_body(uid_hbm, iid_hbm, utab_hbm, itab_hbm, out_hbm,
              uid_v, iid_v, ubuf_v, ibuf_v, out_v, sem_u, sem_i):
    wid = lax.axis_index("s") * NC + lax.axis_index("c")
    base = wid * BPW

    pltpu.sync_copy(uid_hbm.at[pl.ds(base, BPW)], uid_v)
    pltpu.sync_copy(iid_hbm.at[pl.ds(base, BPW)], iid_v)

    lane = lax.iota(jnp.int32, L)

    def chunk_body(b, carry):
        uvec = uid_v[pl.ds(b * CHUNK, CHUNK)]
        ivec = iid_v[pl.ds(b * CHUNK, CHUNK)]
        cu = jnp.right_shift(uvec, 7) * 128
        ci = jnp.right_shift(ivec, 7) * 128
        copies = []
        for j in range(CHUNK):
            cuj = pl.multiple_of(jnp.sum(jnp.where(lane == j, cu, 0)), 128)
            cij = pl.multiple_of(jnp.sum(jnp.where(lane == j, ci, 0)), 128)
            copies.append(pltpu.async_copy(
                utab_hbm.at[:, pl.ds(cuj, 128)], ubuf_v.at[j], sem_u))
            copies.append(pltpu.async_copy(
                itab_hbm.at[:, pl.ds(cij, 128)], ibuf_v.at[j], sem_i))
        for cp in copies:
            cp.wait()

        lu = jnp.bitwise_and(uvec, 127)
        li = jnp.bitwise_and(ivec, 127)
        acc = jnp.zeros((L,), jnp.float32)
        for d in range(D):
            dsplat = jnp.full((L,), d, jnp.int32)
            u = plsc.load_gather(ubuf_v, [lane, dsplat, lu])
            it = plsc.load_gather(ibuf_v, [lane, dsplat, li])
            acc = acc + u * it
        out_v[pl.ds(b * CHUNK, CHUNK)] = 1.0 / (1.0 + jnp.exp(-acc))
        return carry

    lax.fori_loop(0, NCHUNK, chunk_body, 0)
    pltpu.sync_copy(out_v, out_hbm.at[pl.ds(base, BPW)])


def kernel(user_ids, item_ids, source_user, source_item):
    mesh = plsc.VectorSubcoreMesh(
        core_axis_name="c", subcore_axis_name="s",
        num_cores=NC, num_subcores=NS)
    k = pl.kernel(
        _cmf_body,
        out_type=jax.ShapeDtypeStruct((B,), jnp.float32),
        mesh=mesh,
        compiler_params=pltpu.CompilerParams(
            needs_layout_passes=False, use_tc_tiling_on_sc=True),
        scratch_types=[
            pltpu.VMEM((BPW,), jnp.int32),
            pltpu.VMEM((BPW,), jnp.int32),
            pltpu.VMEM((CHUNK, D, 128), jnp.float32),
            pltpu.VMEM((CHUNK, D, 128), jnp.float32),
            pltpu.VMEM((BPW,), jnp.float32),
            pltpu.SemaphoreType.DMA,
            pltpu.SemaphoreType.DMA,
        ],
    )
    return k(user_ids.astype(jnp.int32), item_ids.astype(jnp.int32),
             source_user.T, source_item.T)
